# Initial kernel scaffold; baseline (speedup 1.0000x reference)
import jax
import jax.numpy as jnp
from jax.experimental import pallas as pl


def kernel(coord, feat, params, offset, segment):
    raise NotImplementedError("write your pallas kernel here")


# speedup vs baseline: 4.1914x; 4.1914x over previous
import functools

import jax
import jax.numpy as jnp
from jax import lax
from jax.experimental import pallas as pl
from jax.experimental.pallas import tpu as pltpu
from jax.experimental.pallas import tpu_sc as plsc

_K = 3
_BIG = 3.0e38


def _ceil_to(x, m):
    return -(-x // m) * m


def _knn_body(cq_ref, ct_ref, out_ref, *, n, blk):
    i = pl.program_id(0)
    q = cq_ref[...]
    ct = ct_ref[...]
    npc = ct.shape[1]
    dist = None
    for d in range(3):
        diff = q[:, d:d + 1] - ct[d:d + 1, :]
        sq = diff * diff
        dist = sq if dist is None else dist + sq
    cols = lax.broadcasted_iota(jnp.int32, (blk, npc), 1)
    rown = lax.broadcasted_iota(jnp.int32, (blk, 1), 0) + i * blk
    big = jnp.float32(_BIG)
    dist = jnp.where(cols == rown, big, dist)
    if npc > n:
        dist = jnp.where(cols >= n, big, dist)
    sel = []
    for _ in range(_K):
        m = jnp.min(dist, axis=1, keepdims=True)
        idx = jnp.min(jnp.where(dist == m, cols, jnp.int32(npc)),
                      axis=1, keepdims=True)
        sel.append(idx)
        dist = jnp.where(cols == idx, big, dist)
    c128 = lax.broadcasted_iota(jnp.int32, (blk, 128), 1)
    o = jnp.zeros((blk, 128), jnp.int32)
    for t in range(_K):
        o = jnp.where(c128 == t, sel[t], o)
    out_ref[...] = o


def _knn(coord, n):
    blk = 128
    npr = _ceil_to(n, 256)
    npc = _ceil_to(n, 128)
    cq = jnp.zeros((npr, 8), jnp.float32).at[:n, :3].set(coord)
    ct = jnp.zeros((8, npc), jnp.float32).at[:3, :n].set(coord.T)
    return pl.pallas_call(
        functools.partial(_knn_body, n=n, blk=blk),
        grid=(npr // blk,),
        in_specs=[pl.BlockSpec((blk, 8), lambda i: (i, 0)),
                  pl.BlockSpec((8, npc), lambda i: (0, 0))],
        out_specs=pl.BlockSpec((blk, 128), lambda i: (i, 0)),
        out_shape=jax.ShapeDtypeStruct((npr, 128), jnp.int32),
    )(cq, ct)


def _nearest_body(cq_ref, ct_ref, out_ref, *, nc, blk):
    q = cq_ref[...]
    ct = ct_ref[...]
    npc = ct.shape[1]
    dist = None
    for d in range(3):
        diff = q[:, d:d + 1] - ct[d:d + 1, :]
        sq = diff * diff
        dist = sq if dist is None else dist + sq
    cols = lax.broadcasted_iota(jnp.int32, (blk, npc), 1)
    if npc > nc:
        dist = jnp.where(cols >= nc, jnp.float32(_BIG), dist)
    m = jnp.min(dist, axis=1, keepdims=True)
    idx = jnp.min(jnp.where(dist == m, cols, jnp.int32(npc)),
                  axis=1, keepdims=True)
    out_ref[...] = jnp.broadcast_to(idx, (blk, 128))


def _nearest(fine, coarse, nf, nc):
    blk = 128
    npf = _ceil_to(nf, 256)
    npc = _ceil_to(nc, 128)
    cq = jnp.zeros((npf, 8), jnp.float32).at[:nf, :3].set(fine)
    ct = jnp.zeros((8, npc), jnp.float32).at[:3, :nc].set(coarse.T)
    return pl.pallas_call(
        functools.partial(_nearest_body, nc=nc, blk=blk),
        grid=(npf // blk,),
        in_specs=[pl.BlockSpec((blk, 8), lambda i: (i, 0)),
                  pl.BlockSpec((8, npc), lambda i: (0, 0))],
        out_specs=pl.BlockSpec((blk, 128), lambda i: (i, 0)),
        out_shape=jax.ShapeDtypeStruct((npf, 128), jnp.int32),
    )(cq, ct)


def _sc_gather(table, idx_flat):
    b = idx_flat.shape[0]
    d = table.shape[1]
    info = plsc.get_sparse_core_info()
    nw = info.num_cores * info.num_subcores
    bpw = b // nw
    ch = max(c for c in range(8, min(bpw, 128) + 1, 8) if bpw % c == 0)
    nch = bpw // ch
    idx3 = idx_flat.reshape(nw, nch, ch)
    mesh = plsc.VectorSubcoreMesh(core_axis_name="c", subcore_axis_name="s")

    @functools.partial(
        pl.kernel, mesh=mesh,
        out_type=jax.ShapeDtypeStruct((b, d), jnp.float32),
        scratch_types=[pltpu.VMEM((nch, ch), jnp.int32),
                       pltpu.VMEM((bpw, d), jnp.float32),
                       pltpu.SemaphoreType.DMA],
    )
    def gk(table_hbm, idx_hbm, out_hbm, idx_v, rows_v, sem):
        wid = lax.axis_index("s") * info.num_cores + lax.axis_index("c")
        base = wid * bpw
        pltpu.sync_copy(idx_hbm.at[wid], idx_v)
        copies = [pltpu.async_copy(table_hbm.at[idx_v.at[j]],
                                   rows_v.at[pl.ds(j * ch, ch)], sem)
                  for j in range(nch)]
        for cp in copies:
            cp.wait()
        pltpu.sync_copy(rows_v, out_hbm.at[pl.ds(base, bpw)])

    return gk(table, idx3)


def _bn(x):
    mu = jnp.mean(x, axis=0, keepdims=True)
    var = jnp.mean((x - mu) ** 2, axis=0, keepdims=True)
    return (x - mu) / jnp.sqrt(var + 1e-5)


def _elu(x):
    return jnp.where(x > 0.0, x, jnp.exp(x) - 1.0)


def _mm_body(x_ref, w_ref, o_ref):
    o_ref[...] = jnp.dot(x_ref[...], w_ref[...],
                         preferred_element_type=jnp.float32)


def _mm(x, w):
    return pl.pallas_call(
        _mm_body,
        out_shape=jax.ShapeDtypeStruct((x.shape[0], w.shape[1]), jnp.float32),
    )(x, w)


def _mm_bias_body(x_ref, w_ref, b_ref, o_ref):
    o_ref[...] = (jnp.dot(x_ref[...], w_ref[...],
                          preferred_element_type=jnp.float32) + b_ref[...])


def _mm_bias(x, w, b):
    return pl.pallas_call(
        _mm_bias_body,
        out_shape=jax.ShapeDtypeStruct((x.shape[0], w.shape[1]), jnp.float32),
    )(x, w, b.reshape(1, -1))


def _emb_body(x_ref, w_ref, o_ref):
    h = jnp.dot(x_ref[...], w_ref[...], preferred_element_type=jnp.float32)
    o_ref[...] = jnp.maximum(_bn(h), 0.0)


def _emb(feat, w):
    return pl.pallas_call(
        _emb_body,
        out_shape=jax.ShapeDtypeStruct((feat.shape[0], w.shape[1]),
                                       jnp.float32),
    )(feat, w)


def _mp_post_body(h_ref, g_ref, w_ref, o_ref, *, n, npg):
    h = h_ref[...]
    d = h.shape[1]
    agg = (g_ref[0:n, 0:d] + g_ref[npg:npg + n, 0:d]
           + g_ref[2 * npg:2 * npg + n, 0:d]) / 3.0
    hn = _bn(jnp.dot(h + agg, w_ref[...], preferred_element_type=jnp.float32))
    o_ref[...] = h + _elu(hn)


def _mp_post_skip_body(h_ref, g_ref, w_ref, s_ref, o_ref, *, n, npg):
    h = h_ref[...]
    d = h.shape[1]
    agg = (g_ref[0:n, 0:d] + g_ref[npg:npg + n, 0:d]
           + g_ref[2 * npg:2 * npg + n, 0:d]) / 3.0
    hn = _bn(jnp.dot(h + agg, w_ref[...], preferred_element_type=jnp.float32))
    o_ref[...] = h + _elu(hn) + s_ref[...]


def _pad_cols(w):
    dout = w.shape[-1]
    if dout >= 128:
        return w
    return jnp.zeros(w.shape[:-1] + (128,), jnp.float32).at[..., :dout].set(w)


def _mp_block(h, layers, n, npg, idx_flat, skip=None):
    d = h.shape[1]
    for li, (wm, wu) in enumerate(layers):
        msg = _mm(h, _pad_cols(wm))
        g = _sc_gather(msg, idx_flat)
        last = li == len(layers) - 1
        if last and skip is not None:
            h = pl.pallas_call(
                functools.partial(_mp_post_skip_body, n=n, npg=npg),
                out_shape=jax.ShapeDtypeStruct((n, d), jnp.float32),
            )(h, g, wu, skip)
        else:
            h = pl.pallas_call(
                functools.partial(_mp_post_body, n=n, npg=npg),
                out_shape=jax.ShapeDtypeStruct((n, d), jnp.float32),
            )(h, g, wu)
    return h


def _cls_body(h_ref, w1_ref, b1_ref, w2_ref, o_ref):
    h = (jnp.dot(h_ref[...], w1_ref[...], preferred_element_type=jnp.float32)
         + b1_ref[...])
    h = jnp.maximum(_bn(h), 0.0)
    o_ref[...] = jnp.dot(h, w2_ref[...], preferred_element_type=jnp.float32)


def _cls(h, w1, b1, w2):
    return pl.pallas_call(
        _cls_body,
        out_shape=jax.ShapeDtypeStruct((h.shape[0], w2.shape[1]),
                                       jnp.float32),
    )(h, w1, b1.reshape(1, -1), w2)


def _mp_idx(knn_out, n):
    npg = knn_out.shape[0]
    r = jnp.arange(npg, dtype=jnp.int32)
    cols = [jnp.where(r < n, knn_out[:, t], 0) for t in range(_K)]
    return jnp.concatenate(cols, axis=0)


def _assign_idx(near_out, nf):
    npf = near_out.shape[0]
    r = jnp.arange(npf, dtype=jnp.int32)
    return jnp.where(r < nf, near_out[:, 0], 0)


def kernel(coord, feat, params, offset, segment):
    del offset, segment
    n0 = coord.shape[0]
    ns = [n0]
    for _ in range(4):
        ns.append(ns[-1] // 4)
    cs = [coord]
    for l in range(4):
        m = ns[l + 1]
        cs.append(cs[l][0:4 * m:4])

    knn_outs = [_knn(cs[l], ns[l]) for l in range(5)]
    idx_flats = [_mp_idx(knn_outs[l], ns[l]) for l in range(5)]
    npgs = [knn_outs[l].shape[0] for l in range(5)]
    assign_idx = [_assign_idx(_nearest(cs[l], cs[l + 1], ns[l], ns[l + 1]),
                              ns[l]) for l in range(4)]

    h = _emb(feat, params['W_emb'])
    hs = []
    for l in range(4):
        h = _mp_block(h, params['mp_down'][l], ns[l], npgs[l], idx_flats[l])
        hs.append(h)
        wd, bd = params['down'][l]
        m = ns[l + 1]
        h = _mm_bias(h[0:4 * m:4], wd, bd)

    h = _mp_block(h, params['mp_bot'], ns[4], npgs[4], idx_flats[4])

    for li in range(4):
        up = 4 - li
        w, b = params['up_proj'][li]
        proj = _mm_bias(h, _pad_cols(w), _pad_cols(b.reshape(1, -1))[0])
        g = _sc_gather(proj, assign_idx[up - 1])
        h = g[:ns[up - 1], :w.shape[1]]
        h = _mp_block(h, params['mp_up'][li], ns[up - 1], npgs[up - 1],
                      idx_flats[up - 1], skip=hs[up - 1])

    w1, b1, w2 = params['cls']
    return _cls(h, w1, b1, w2)


# speedup vs baseline: 5.1330x; 1.2246x over previous
import functools

import jax
import jax.numpy as jnp
from jax import lax
from jax.experimental import pallas as pl
from jax.experimental.pallas import tpu as pltpu
from jax.experimental.pallas import tpu_sc as plsc

_K = 3
_BIG = 3.0e38


def _ceil_to(x, m):
    return -(-x // m) * m


def _knn_body(cq_ref, ct_ref, out_ref, *, n, blk):
    i = pl.program_id(0)
    q = cq_ref[...]
    ct = ct_ref[...]
    npc = ct.shape[1]
    dist = None
    for d in range(3):
        diff = q[:, d:d + 1] - ct[d:d + 1, :]
        sq = diff * diff
        dist = sq if dist is None else dist + sq
    cols = lax.broadcasted_iota(jnp.int32, (blk, npc), 1)
    rown = lax.broadcasted_iota(jnp.int32, (blk, 1), 0) + i * blk
    big = jnp.float32(_BIG)
    dist = jnp.where(cols == rown, big, dist)
    if npc > n:
        dist = jnp.where(cols >= n, big, dist)
    sel = []
    for _ in range(_K):
        m = jnp.min(dist, axis=1, keepdims=True)
        idx = jnp.min(jnp.where(dist == m, cols, jnp.int32(npc)),
                      axis=1, keepdims=True)
        sel.append(idx)
        dist = jnp.where(cols == idx, big, dist)
    c128 = lax.broadcasted_iota(jnp.int32, (blk, 128), 1)
    o = jnp.zeros((blk, 128), jnp.int32)
    for t in range(_K):
        o = jnp.where(c128 == t, sel[t], o)
    out_ref[...] = o


def _knn(coord, n):
    blk = 128
    npr = _ceil_to(n, 256)
    npc = _ceil_to(n, 128)
    cq = jnp.zeros((npr, 8), jnp.float32).at[:n, :3].set(coord)
    ct = jnp.zeros((8, npc), jnp.float32).at[:3, :n].set(coord.T)
    return pl.pallas_call(
        functools.partial(_knn_body, n=n, blk=blk),
        grid=(npr // blk,),
        in_specs=[pl.BlockSpec((blk, 8), lambda i: (i, 0)),
                  pl.BlockSpec((8, npc), lambda i: (0, 0))],
        out_specs=pl.BlockSpec((blk, 128), lambda i: (i, 0)),
        out_shape=jax.ShapeDtypeStruct((npr, 128), jnp.int32),
    )(cq, ct)


def _nearest_body(cq_ref, ct_ref, out_ref, *, nc, blk):
    q = cq_ref[...]
    ct = ct_ref[...]
    npc = ct.shape[1]
    dist = None
    for d in range(3):
        diff = q[:, d:d + 1] - ct[d:d + 1, :]
        sq = diff * diff
        dist = sq if dist is None else dist + sq
    cols = lax.broadcasted_iota(jnp.int32, (blk, npc), 1)
    if npc > nc:
        dist = jnp.where(cols >= nc, jnp.float32(_BIG), dist)
    m = jnp.min(dist, axis=1, keepdims=True)
    idx = jnp.min(jnp.where(dist == m, cols, jnp.int32(npc)),
                  axis=1, keepdims=True)
    out_ref[...] = jnp.broadcast_to(idx, (blk, 128))


def _nearest(fine, coarse, nf, nc):
    blk = 128
    npf = _ceil_to(nf, 256)
    npc = _ceil_to(nc, 128)
    cq = jnp.zeros((npf, 8), jnp.float32).at[:nf, :3].set(fine)
    ct = jnp.zeros((8, npc), jnp.float32).at[:3, :nc].set(coarse.T)
    return pl.pallas_call(
        functools.partial(_nearest_body, nc=nc, blk=blk),
        grid=(npf // blk,),
        in_specs=[pl.BlockSpec((blk, 8), lambda i: (i, 0)),
                  pl.BlockSpec((8, npc), lambda i: (0, 0))],
        out_specs=pl.BlockSpec((blk, 128), lambda i: (i, 0)),
        out_shape=jax.ShapeDtypeStruct((npf, 128), jnp.int32),
    )(cq, ct)


def _sc_gather(table, idx_flat):
    b = idx_flat.shape[0]
    d = table.shape[1]
    info = plsc.get_sparse_core_info()
    nw = info.num_cores * info.num_subcores
    bpw = b // nw
    ch = max(c for c in range(8, min(bpw, 128) + 1, 8) if bpw % c == 0)
    nch = bpw // ch
    idx3 = idx_flat.reshape(nw, nch, ch)
    mesh = plsc.VectorSubcoreMesh(core_axis_name="c", subcore_axis_name="s")

    @functools.partial(
        pl.kernel, mesh=mesh,
        out_type=jax.ShapeDtypeStruct((b, d), jnp.float32),
        scratch_types=[pltpu.VMEM((nch, ch), jnp.int32),
                       pltpu.VMEM((bpw, d), jnp.float32),
                       pltpu.SemaphoreType.DMA],
    )
    def gk(table_hbm, idx_hbm, out_hbm, idx_v, rows_v, sem):
        wid = lax.axis_index("s") * info.num_cores + lax.axis_index("c")
        base = wid * bpw
        pltpu.sync_copy(idx_hbm.at[wid], idx_v)
        copies = [pltpu.async_copy(table_hbm.at[idx_v.at[j]],
                                   rows_v.at[pl.ds(j * ch, ch)], sem)
                  for j in range(nch)]
        for cp in copies:
            cp.wait()
        pltpu.sync_copy(rows_v, out_hbm.at[pl.ds(base, bpw)])

    return gk(table, idx3)


def _bn(x):
    mu = jnp.mean(x, axis=0, keepdims=True)
    var = jnp.mean((x - mu) ** 2, axis=0, keepdims=True)
    return (x - mu) / jnp.sqrt(var + 1e-5)


def _elu(x):
    return jnp.where(x > 0.0, x, jnp.exp(x) - 1.0)


def _mm_body(x_ref, w_ref, o_ref):
    o_ref[...] = jnp.dot(x_ref[...], w_ref[...],
                         preferred_element_type=jnp.float32)


def _mm(x, w):
    return pl.pallas_call(
        _mm_body,
        out_shape=jax.ShapeDtypeStruct((x.shape[0], w.shape[1]), jnp.float32),
    )(x, w)


def _mm_bias_body(x_ref, w_ref, b_ref, o_ref):
    o_ref[...] = (jnp.dot(x_ref[...], w_ref[...],
                          preferred_element_type=jnp.float32) + b_ref[...])


def _mm_bias(x, w, b):
    return pl.pallas_call(
        _mm_bias_body,
        out_shape=jax.ShapeDtypeStruct((x.shape[0], w.shape[1]), jnp.float32),
    )(x, w, b.reshape(1, -1))


def _emb_body(x_ref, w_ref, o_ref):
    h = jnp.dot(x_ref[...], w_ref[...], preferred_element_type=jnp.float32)
    o_ref[...] = jnp.maximum(_bn(h), 0.0)


def _emb(feat, w):
    return pl.pallas_call(
        _emb_body,
        out_shape=jax.ShapeDtypeStruct((feat.shape[0], w.shape[1]),
                                       jnp.float32),
    )(feat, w)


def _mp_post_body(h_ref, g_ref, w_ref, o_ref, *, n, npg):
    h = h_ref[...]
    d = h.shape[1]
    agg = (g_ref[0:n, 0:d] + g_ref[npg:npg + n, 0:d]
           + g_ref[2 * npg:2 * npg + n, 0:d]) / 3.0
    hn = _bn(jnp.dot(h + agg, w_ref[...], preferred_element_type=jnp.float32))
    o_ref[...] = h + _elu(hn)


def _mp_post_skip_body(h_ref, g_ref, w_ref, s_ref, o_ref, *, n, npg):
    h = h_ref[...]
    d = h.shape[1]
    agg = (g_ref[0:n, 0:d] + g_ref[npg:npg + n, 0:d]
           + g_ref[2 * npg:2 * npg + n, 0:d]) / 3.0
    hn = _bn(jnp.dot(h + agg, w_ref[...], preferred_element_type=jnp.float32))
    o_ref[...] = h + _elu(hn) + s_ref[...]


def _pad_cols(w):
    dout = w.shape[-1]
    if dout >= 128:
        return w
    return jnp.zeros(w.shape[:-1] + (128,), jnp.float32).at[..., :dout].set(w)


def _mp_block_sc(h, layers, n, npg, idx_flat, skip=None):
    d = h.shape[1]
    for li, (wm, wu) in enumerate(layers):
        msg = _mm(h, _pad_cols(wm))
        g = _sc_gather(msg, idx_flat)
        last = li == len(layers) - 1
        if last and skip is not None:
            h = pl.pallas_call(
                functools.partial(_mp_post_skip_body, n=n, npg=npg),
                out_shape=jax.ShapeDtypeStruct((n, d), jnp.float32),
            )(h, g, wu, skip)
        else:
            h = pl.pallas_call(
                functools.partial(_mp_post_body, n=n, npg=npg),
                out_shape=jax.ShapeDtypeStruct((n, d), jnp.float32),
            )(h, g, wu)
    return h


def _adj_onehot(nn_vals, n):
    cols = lax.broadcasted_iota(jnp.int32, (n, n), 1)
    a = None
    for t in range(_K):
        hit = jnp.where(cols == nn_vals[:, t:t + 1], 1.0, 0.0)
        a = hit if a is None else a + hit
    return a


def _mp_layers(h, a, w_refs):
    for wm_ref, wu_ref in w_refs:
        msg = jnp.dot(h, wm_ref[...], preferred_element_type=jnp.float32)
        agg = jnp.dot(a, msg, preferred_element_type=jnp.float32) / 3.0
        hn = _bn(jnp.dot(h + agg, wu_ref[...],
                         preferred_element_type=jnp.float32))
        h = h + _elu(hn)
    return h


def _mp_tc_body(h_ref, nn_ref, wm1_ref, wu1_ref, wm2_ref, wu2_ref, o_ref,
                *, n):
    a = _adj_onehot(nn_ref[0:n, :], n)
    h = _mp_layers(h_ref[...], a,
                   ((wm1_ref, wu1_ref), (wm2_ref, wu2_ref)))
    o_ref[...] = h


def _mp_tc(h, nn_out, layers, n):
    (wm1, wu1), (wm2, wu2) = layers
    return pl.pallas_call(
        functools.partial(_mp_tc_body, n=n),
        out_shape=jax.ShapeDtypeStruct(h.shape, jnp.float32),
    )(h, nn_out, wm1, wu1, wm2, wu2)


def _up_tc_body(h_ref, wp_ref, bp_ref, an_ref, nn_ref, wm1_ref, wu1_ref,
                wm2_ref, wu2_ref, s_ref, o_ref, *, nf, nc):
    proj = (jnp.dot(h_ref[...], wp_ref[...],
                    preferred_element_type=jnp.float32) + bp_ref[...])
    cols = lax.broadcasted_iota(jnp.int32, (nf, nc), 1)
    a2 = jnp.where(cols == an_ref[0:nf, 0:1], 1.0, 0.0)
    h = jnp.dot(a2, proj, preferred_element_type=jnp.float32)
    a = _adj_onehot(nn_ref[0:nf, :], nf)
    h = _mp_layers(h, a, ((wm1_ref, wu1_ref), (wm2_ref, wu2_ref)))
    o_ref[...] = h + s_ref[...]


def _up_tc(h, wp, bp, an_out, nn_out, layers, skip, nf, nc):
    (wm1, wu1), (wm2, wu2) = layers
    return pl.pallas_call(
        functools.partial(_up_tc_body, nf=nf, nc=nc),
        out_shape=jax.ShapeDtypeStruct((nf, wp.shape[1]), jnp.float32),
    )(h, wp, bp.reshape(1, -1), an_out, nn_out, wm1, wu1, wm2, wu2, skip)


def _proj_assign_body(h_ref, wp_ref, bp_ref, an_ref, o_ref, *, nf, nc):
    proj = (jnp.dot(h_ref[...], wp_ref[...],
                    preferred_element_type=jnp.float32) + bp_ref[...])
    cols = lax.broadcasted_iota(jnp.int32, (nf, nc), 1)
    a2 = jnp.where(cols == an_ref[0:nf, 0:1], 1.0, 0.0)
    o_ref[...] = jnp.dot(a2, proj, preferred_element_type=jnp.float32)


def _proj_assign(h, wp, bp, an_out, nf, nc):
    return pl.pallas_call(
        functools.partial(_proj_assign_body, nf=nf, nc=nc),
        out_shape=jax.ShapeDtypeStruct((nf, wp.shape[1]), jnp.float32),
    )(h, wp, bp.reshape(1, -1), an_out)


def _cls_body(h_ref, w1_ref, b1_ref, w2_ref, o_ref):
    h = (jnp.dot(h_ref[...], w1_ref[...], preferred_element_type=jnp.float32)
         + b1_ref[...])
    h = jnp.maximum(_bn(h), 0.0)
    o_ref[...] = jnp.dot(h, w2_ref[...], preferred_element_type=jnp.float32)


def _cls(h, w1, b1, w2):
    return pl.pallas_call(
        _cls_body,
        out_shape=jax.ShapeDtypeStruct((h.shape[0], w2.shape[1]),
                                       jnp.float32),
    )(h, w1, b1.reshape(1, -1), w2)


def _mp_idx(knn_out, n):
    npg = knn_out.shape[0]
    r = jnp.arange(npg, dtype=jnp.int32)
    cols = [jnp.where(r < n, knn_out[:, t], 0) for t in range(_K)]
    return jnp.concatenate(cols, axis=0)


def _assign_idx(near_out, nf):
    npf = near_out.shape[0]
    r = jnp.arange(npf, dtype=jnp.int32)
    return jnp.where(r < nf, near_out[:, 0], 0)


def kernel(coord, feat, params, offset, segment):
    del offset, segment
    n0 = coord.shape[0]
    ns = [n0]
    for _ in range(4):
        ns.append(ns[-1] // 4)
    cs = [coord]
    for l in range(4):
        m = ns[l + 1]
        cs.append(cs[l][0:4 * m:4])

    sc_level = [n > 1000 for n in ns]
    knn_outs = [_knn(cs[l], ns[l]) for l in range(5)]
    idx_flats = [_mp_idx(knn_outs[l], ns[l]) if sc_level[l] else None
                 for l in range(5)]
    npgs = [knn_outs[l].shape[0] for l in range(5)]
    near_outs = [_nearest(cs[l], cs[l + 1], ns[l], ns[l + 1])
                 for l in range(4)]

    h = _emb(feat, params['W_emb'])
    hs = []
    for l in range(4):
        if sc_level[l]:
            h = _mp_block_sc(h, params['mp_down'][l], ns[l], npgs[l],
                             idx_flats[l])
        else:
            h = _mp_tc(h, knn_outs[l], params['mp_down'][l], ns[l])
        hs.append(h)
        wd, bd = params['down'][l]
        m = ns[l + 1]
        h = _mm_bias(h[0:4 * m:4], wd, bd)

    h = _mp_tc(h, knn_outs[4], params['mp_bot'], ns[4])

    for li in range(4):
        up = 4 - li
        w, b = params['up_proj'][li]
        nf, nc = ns[up - 1], ns[up]
        if sc_level[up - 1]:
            if nf > 4000:
                proj = _mm_bias(h, _pad_cols(w), _pad_cols(b.reshape(1, -1))[0])
                g = _sc_gather(proj, _assign_idx(near_outs[up - 1], nf))
                h = g[:nf, :w.shape[1]]
            else:
                h = _proj_assign(h, w, b, near_outs[up - 1], nf, nc)
            h = _mp_block_sc(h, params['mp_up'][li], nf, npgs[up - 1],
                             idx_flats[up - 1], skip=hs[up - 1])
        else:
            h = _up_tc(h, w, b, near_outs[up - 1], knn_outs[up - 1],
                       params['mp_up'][li], hs[up - 1], nf, nc)

    w1, b1, w2 = params['cls']
    return _cls(h, w1, b1, w2)
